# baseline (speedup 1.0000x reference)
import functools

import jax
import jax.numpy as jnp
from jax import lax
from jax.experimental import pallas as pl
from jax.experimental.pallas import tpu as pltpu
from jax.experimental.pallas import tpu_sc as plsc

EMB_DIM = 16
CHUNK_COLS = 1280


@functools.cache
def _build_broadcast(n_edges: int, emb_dim: int):
    info = plsc.get_sparse_core_info()
    num_workers = info.num_cores * info.num_subcores
    lanes = info.num_lanes
    assert n_edges % CHUNK_COLS == 0
    n_chunks = n_edges // CHUNK_COLS

    mesh = plsc.VectorSubcoreMesh(core_axis_name="c", subcore_axis_name="s")

    @functools.partial(
        pl.kernel,
        mesh=mesh,
        out_type=jax.ShapeDtypeStruct((emb_dim, n_edges), jnp.float32),
        scratch_types=[
            pltpu.VMEM((emb_dim, lanes), jnp.float32),
            pltpu.VMEM((emb_dim, CHUNK_COLS), jnp.float32),
            pltpu.SemaphoreType.DMA,
        ],
    )
    def bcast(splat_hbm, out_hbm, sv, buf, sem):
        wid = lax.axis_index("s") * info.num_cores + lax.axis_index("c")
        pltpu.sync_copy(splat_hbm, sv)

        def fill(c, _):
            srow = sv[c]

            def fill_row(j, _):
                buf[c, pl.ds(j * lanes, lanes)] = srow
                return 0

            lax.fori_loop(0, CHUNK_COLS // lanes, fill_row, 0)
            return 0

        lax.fori_loop(0, emb_dim, fill, 0)

        n_mine = (n_chunks - 1 - wid) // num_workers + 1

        def start(i, _):
            base = (wid + i * num_workers) * CHUNK_COLS
            pltpu.make_async_copy(
                buf, out_hbm.at[:, pl.ds(base, CHUNK_COLS)], sem
            ).start()
            return 0

        def drain(i, _):
            pltpu.make_async_copy(
                buf, out_hbm.at[:, pl.ds(wid * CHUNK_COLS, CHUNK_COLS)], sem
            ).wait()
            return 0

        lax.fori_loop(0, n_mine, start, 0)
        lax.fori_loop(0, n_mine, drain, 0)

    return bcast


def kernel(batch, W):
    n_edges = batch.shape[0]
    splat = jnp.broadcast_to(W.reshape(EMB_DIM, 1), (EMB_DIM, EMB_DIM))
    cols = _build_broadcast(n_edges, EMB_DIM)(splat)
    return cols.T


# speedup vs baseline: 1.0774x; 1.0774x over previous
import functools

import jax
import jax.numpy as jnp
from jax import lax
from jax.experimental import pallas as pl
from jax.experimental.pallas import tpu as pltpu
from jax.experimental.pallas import tpu_sc as plsc

EMB_DIM = 16
CHUNK_COLS = 1280


@functools.cache
def _build_broadcast(n_edges: int, emb_dim: int):
    info = plsc.get_sparse_core_info()
    num_workers = info.num_cores * info.num_subcores
    lanes = info.num_lanes
    assert n_edges % CHUNK_COLS == 0
    n_chunks = n_edges // CHUNK_COLS

    mesh = plsc.VectorSubcoreMesh(core_axis_name="c", subcore_axis_name="s")

    @functools.partial(
        pl.kernel,
        mesh=mesh,
        out_type=jax.ShapeDtypeStruct((emb_dim, n_edges), jnp.float32),
        scratch_types=[
            pltpu.VMEM((emb_dim, lanes), jnp.float32),
            pltpu.VMEM((emb_dim, CHUNK_COLS), jnp.float32),
            pltpu.SemaphoreType.DMA,
        ],
    )
    def bcast(splat_hbm, out_hbm, sv, buf, sem):
        wid = lax.axis_index("s") * info.num_cores + lax.axis_index("c")
        pltpu.sync_copy(splat_hbm, sv)
        splats = [sv[c] for c in range(emb_dim)]

        def fill(j, _):
            for c in range(emb_dim):
                buf[c, pl.ds(j * lanes, lanes)] = splats[c]
            return 0

        lax.fori_loop(0, CHUNK_COLS // lanes, fill, 0)

        n_mine = (n_chunks - 1 - wid) // num_workers + 1

        def start(i, _):
            base = (wid + i * num_workers) * CHUNK_COLS
            pltpu.make_async_copy(
                buf, out_hbm.at[:, pl.ds(base, CHUNK_COLS)], sem
            ).start()
            return 0

        def drain(i, _):
            pltpu.make_async_copy(
                buf, out_hbm.at[:, pl.ds(wid * CHUNK_COLS, CHUNK_COLS)], sem
            ).wait()
            return 0

        lax.fori_loop(0, n_mine, start, 0)
        lax.fori_loop(0, n_mine, drain, 0)

    return bcast


def kernel(batch, W):
    n_edges = batch.shape[0]
    splat = jnp.broadcast_to(W.reshape(EMB_DIM, 1), (EMB_DIM, EMB_DIM))
    cols = _build_broadcast(n_edges, EMB_DIM)(splat)
    return cols.T
